# Initial kernel scaffold; baseline (speedup 1.0000x reference)
import jax
import jax.numpy as jnp
from jax.experimental import pallas as pl


def kernel(x, transitions, transition_probs, initial_probs):
    raise NotImplementedError("write your pallas kernel here")


# speedup vs baseline: 11.6090x; 11.6090x over previous
import functools

import jax
import jax.numpy as jnp
from jax import lax
from jax.experimental import pallas as pl
from jax.experimental.pallas import tpu as pltpu
from jax.experimental.pallas import tpu_sc as plsc

L = 16


def _fwd_body(num_states, num_pdfs, num_trans, seq_len,
              x_hbm, packed_hbm, w_hbm, init_hbm, c_hbm,
              packed_v, w_v, alpha_v, new_v, xrow_v, px_v, c_v):
    b = lax.axis_index("s") * 2 + lax.axis_index("c")

    pltpu.sync_copy(packed_hbm, packed_v)
    pltpu.sync_copy(w_hbm, w_v)
    pltpu.sync_copy(init_hbm, alpha_v)

    zeros = jnp.zeros((L,), jnp.float32)

    @pl.loop(0, num_states // L)
    def _(j):
        new_v[pl.ds(j * L, L)] = zeros

    @pl.loop(0, seq_len)
    def _(t):
        pltpu.sync_copy(x_hbm.at[b, t], xrow_v)

        @pl.loop(0, num_pdfs // L)
        def _(j):
            px_v[pl.ds(j * L, L)] = jnp.exp(xrow_v[pl.ds(j * L, L)])

        @pl.loop(0, num_trans // L)
        def _(i):
            base = i * L
            pk = packed_v[pl.ds(base, L)]
            wv = w_v[pl.ds(base, L)]
            s = pk & (num_states - 1)
            d = (pk >> 11) & (num_states - 1)
            p = pk >> 22
            a = plsc.load_gather(alpha_v, [s])
            e = plsc.load_gather(px_v, [p])
            plsc.addupdate_scatter(new_v, [d], a * wv * e)

        acc = lax.fori_loop(
            0, num_states // L,
            lambda j, a: a + new_v[pl.ds(j * L, L)], zeros)
        cvec = jnp.full((L,), jnp.sum(acc), jnp.float32)
        c_v[pl.ds(t * L, L)] = cvec
        rvec = jnp.ones((L,), jnp.float32) / cvec

        @pl.loop(0, num_states // L)
        def _(j):
            v = new_v[pl.ds(j * L, L)]
            alpha_v[pl.ds(j * L, L)] = v * rvec
            new_v[pl.ds(j * L, L)] = zeros

    pltpu.sync_copy(c_v, c_hbm.at[b])


def _logsum_body(c_ref, o_ref):
    o_ref[...] = jnp.sum(jnp.log(c_ref[...])).reshape(1, 1)


def kernel(x, transitions, transition_probs, initial_probs):
    B, T, P = x.shape
    S = initial_probs.shape[0]
    NT = transitions.shape[0]

    src = transitions[:, 0]
    dst = transitions[:, 1]
    pdf = transitions[:, 2]
    packed = (src | (dst << 11) | (pdf << 22)).astype(jnp.int32)

    mesh = plsc.VectorSubcoreMesh(core_axis_name="c", subcore_axis_name="s")
    fwd = pl.kernel(
        functools.partial(_fwd_body, S, P, NT, T),
        out_type=jax.ShapeDtypeStruct((B, T * L), jnp.float32),
        mesh=mesh,
        compiler_params=pltpu.CompilerParams(needs_layout_passes=False),
        scratch_types=[
            pltpu.VMEM((NT,), jnp.int32),
            pltpu.VMEM((NT,), jnp.float32),
            pltpu.VMEM((S,), jnp.float32),
            pltpu.VMEM((S,), jnp.float32),
            pltpu.VMEM((P,), jnp.float32),
            pltpu.VMEM((P,), jnp.float32),
            pltpu.VMEM((T * L,), jnp.float32),
        ],
    )
    c_mat = fwd(x, packed, transition_probs, initial_probs)

    tot = pl.pallas_call(
        _logsum_body,
        out_shape=jax.ShapeDtypeStruct((1, 1), jnp.float32),
    )(c_mat[:, ::L])
    return tot[0, 0] / B
